# baseline (speedup 1.0000x reference)
import functools

import jax
import jax.numpy as jnp
from jax import lax
from jax.experimental import pallas as pl
from jax.experimental.pallas import tpu as pltpu
from jax.experimental.pallas import tpu_sc as plsc

_NC, _NS = 2, 16
_NW = _NC * _NS
_CH = 80
_DP = 384


def _gather_sc(table, idx):
    vocab, d = table.shape
    assert d == 300
    bt = idx.shape[0]
    b_per_w = bt // _NW
    n_chunks = b_per_w // _CH
    assert b_per_w % _CH == 0 and n_chunks >= 4

    mesh = plsc.VectorSubcoreMesh(core_axis_name="c", subcore_axis_name="s")

    @functools.partial(
        pl.kernel,
        out_type=jax.ShapeDtypeStruct((bt, _DP), jnp.float32),
        mesh=mesh,
        scratch_types=[
            pltpu.VMEM((b_per_w,), jnp.int32),
            pltpu.VMEM((4, _CH, _DP), jnp.float32),
            pltpu.SemaphoreType.DMA,
            pltpu.SemaphoreType.DMA,
        ],
    )
    def k(table_hbm, idx_hbm, out_hbm, idx_v, rows_v, gsem, wsem):
        wid = lax.axis_index("s") * _NC + lax.axis_index("c")
        base = wid * b_per_w
        pltpu.sync_copy(idx_hbm.at[pl.ds(base, b_per_w)], idx_v)

        def start_gather(j, slot):
            ids = idx_v.at[pl.ds(j * _CH, _CH)]
            off0 = pl.multiple_of(jnp.full((), 0, jnp.int32), 128)
            pltpu.async_copy(table_hbm.at[ids, pl.ds(off0, _DP)],
                             rows_v.at[slot], gsem)

        def wait_gather(slot):
            pltpu.make_async_copy(
                table_hbm.at[idx_v.at[pl.ds(0, _CH)], pl.ds(0, _DP)],
                rows_v.at[slot], gsem).wait()

        def wait_writeback(slot):
            pltpu.make_async_copy(
                rows_v.at[slot], out_hbm.at[pl.ds(base, _CH)], wsem).wait()

        for p in range(3):
            start_gather(p, p)

        def body(j, carry):
            slot = lax.rem(j, 4)
            pslot = lax.rem(j + 3, 4)

            @pl.when(jnp.logical_and(j + 3 < n_chunks, j >= 1))
            def _():
                wait_writeback(pslot)

            @pl.when(j + 3 < n_chunks)
            def _():
                start_gather(j + 3, pslot)

            wait_gather(slot)
            pltpu.async_copy(rows_v.at[slot],
                             out_hbm.at[pl.ds(base + j * _CH, _CH)], wsem)
            return carry

        lax.fori_loop(0, n_chunks, body, 0)
        for p in range(4):
            wait_writeback(p)

    return k(table, idx)


def _project_tc(emb, w, b2d, batch, hist):
    m, kdim = emb.shape
    n = w.shape[1]
    kw = w.shape[0]
    bb = 32
    assert batch % bb == 0 and m == batch * hist

    def mk(e_ref, w_ref, b_ref, o_ref):
        w16 = w_ref[...].astype(jnp.bfloat16)
        for t in range(bb):
            o_ref[t] = (
                jnp.dot(
                    e_ref[pl.ds(t * hist, hist), pl.ds(0, kw)].astype(jnp.bfloat16),
                    w16, preferred_element_type=jnp.float32)
                + b_ref[...]
            )

    return pl.pallas_call(
        mk,
        grid=(batch // bb,),
        in_specs=[
            pl.BlockSpec((bb * hist, kdim), lambda i: (i, 0)),
            pl.BlockSpec((kw, n), lambda i: (0, 0)),
            pl.BlockSpec((1, n), lambda i: (0, 0)),
        ],
        out_specs=pl.BlockSpec((bb, hist, n), lambda i: (i, 0, 0)),
        out_shape=jax.ShapeDtypeStruct((batch, hist, n), jnp.float32),
    )(emb, w, b2d)


def kernel(x, glove_table, W, b):
    batch, hist = x.shape
    n = W.shape[1]
    idx = x.astype(jnp.int32).reshape(-1)
    emb = _gather_sc(glove_table, idx)
    return _project_tc(emb, W, b.reshape(1, n), batch, hist)


# speedup vs baseline: 1.0064x; 1.0064x over previous
import functools

import jax
import jax.numpy as jnp
from jax import lax
from jax.experimental import pallas as pl
from jax.experimental.pallas import tpu as pltpu
from jax.experimental.pallas import tpu_sc as plsc

_NC, _NS = 2, 16
_NW = _NC * _NS
_CH = 80
_DP = 384


def _gather_sc(table, idx):
    vocab, d = table.shape
    assert d == 300
    bt = idx.shape[0]
    b_per_w = bt // _NW
    n_chunks = b_per_w // _CH
    assert b_per_w % _CH == 0 and n_chunks >= 4

    mesh = plsc.VectorSubcoreMesh(core_axis_name="c", subcore_axis_name="s")

    @functools.partial(
        pl.kernel,
        out_type=jax.ShapeDtypeStruct((bt, _DP), jnp.float32),
        mesh=mesh,
        scratch_types=[
            pltpu.VMEM((b_per_w,), jnp.int32),
            pltpu.VMEM((4, _CH, _DP), jnp.float32),
            pltpu.SemaphoreType.DMA,
            pltpu.SemaphoreType.DMA,
        ],
    )
    def k(table_hbm, idx_hbm, out_hbm, idx_v, rows_v, gsem, wsem):
        wid = lax.axis_index("s") * _NC + lax.axis_index("c")
        base = wid * b_per_w
        pltpu.sync_copy(idx_hbm.at[pl.ds(base, b_per_w)], idx_v)

        def start_gather(j, slot):
            ids = idx_v.at[pl.ds(j * _CH, _CH)]
            off0 = pl.multiple_of(jnp.full((), 0, jnp.int32), 128)
            pltpu.async_copy(table_hbm.at[ids, pl.ds(off0, _DP)],
                             rows_v.at[slot], gsem)

        def wait_gather(slot):
            pltpu.make_async_copy(
                table_hbm.at[idx_v.at[pl.ds(0, _CH)], pl.ds(0, _DP)],
                rows_v.at[slot], gsem).wait()

        def wait_writeback(slot):
            pltpu.make_async_copy(
                rows_v.at[slot], out_hbm.at[pl.ds(base, _CH)], wsem).wait()

        for p in range(3):
            start_gather(p, p)

        def body(j, carry):
            slot = lax.rem(j, 4)
            pslot = lax.rem(j + 3, 4)

            @pl.when(jnp.logical_and(j + 3 < n_chunks, j >= 1))
            def _():
                wait_writeback(pslot)

            @pl.when(j + 3 < n_chunks)
            def _():
                start_gather(j + 3, pslot)

            wait_gather(slot)
            pltpu.async_copy(rows_v.at[slot],
                             out_hbm.at[pl.ds(base + j * _CH, _CH)], wsem)
            return carry

        lax.fori_loop(0, n_chunks, body, 0)
        for p in range(4):
            wait_writeback(p)

    return k(table, idx)


def _project_tc(emb, w, b2d, batch, hist):
    m, kdim = emb.shape
    n = w.shape[1]
    kw = w.shape[0]
    bb = 64
    assert batch % bb == 0 and m == batch * hist

    def mk(e_ref, w_ref, b_ref, o_ref):
        w16 = w_ref[...].astype(jnp.bfloat16)
        for t in range(bb):
            o_ref[t] = (
                jnp.dot(
                    e_ref[pl.ds(t * hist, hist), pl.ds(0, kw)].astype(jnp.bfloat16),
                    w16, preferred_element_type=jnp.float32)
                + b_ref[...]
            )

    return pl.pallas_call(
        mk,
        grid=(batch // bb,),
        in_specs=[
            pl.BlockSpec((bb * hist, kdim), lambda i: (i, 0)),
            pl.BlockSpec((kw, n), lambda i: (0, 0)),
            pl.BlockSpec((1, n), lambda i: (0, 0)),
        ],
        out_specs=pl.BlockSpec((bb, hist, n), lambda i: (i, 0, 0)),
        out_shape=jax.ShapeDtypeStruct((batch, hist, n), jnp.float32),
    )(emb, w, b2d)


def kernel(x, glove_table, W, b):
    batch, hist = x.shape
    n = W.shape[1]
    idx = x.astype(jnp.int32).reshape(-1)
    emb = _gather_sc(glove_table, idx)
    return _project_tc(emb, W, b.reshape(1, n), batch, hist)
